# baseline (speedup 1.0000x reference)
import functools

import jax
import jax.numpy as jnp
from jax import lax
from jax.experimental import pallas as pl
from jax.experimental.pallas import tpu as pltpu
from jax.experimental.pallas import tpu_sc as plsc

T = 2048
D = 768
FF = 1024
E = 16
NP = D // 128
TB = 128
G = T // TB + E
P = G * TB
CH = 256
NW = 32
CHK = T // NW


def _router_body(x_ref, gw_ref, gb_ref, pos_ref, eb_ref, na_ref, xp_ref,
                 oh_ref, cs_ref):
    x = x_ref[...]
    for j in range(NP):
        xp_ref[j] = x[:, j * 128:(j + 1) * 128]
    scores = jnp.dot(x, gw_ref[...],
                     preferred_element_type=jnp.float32) + gb_ref[...]
    e_iota = lax.broadcasted_iota(jnp.int32, (T, E), 1)
    m = jnp.max(scores, axis=1, keepdims=True)
    eid = jnp.min(jnp.where(scores >= m, e_iota, E), axis=1, keepdims=True)
    oh_ref[...] = (e_iota == eid).astype(jnp.float32)

    tri = (lax.broadcasted_iota(jnp.int32, (CH, CH), 0) >=
           lax.broadcasted_iota(jnp.int32, (CH, CH), 1)).astype(jnp.float32)

    def body(i, carry):
        blk = oh_ref[pl.ds(i * CH, CH), :]
        cs = jnp.dot(tri, blk, preferred_element_type=jnp.float32) + carry
        cs_ref[pl.ds(i * CH, CH), :] = cs
        return cs[CH - 1:CH, :]

    counts = lax.fori_loop(0, T // CH, body,
                           jnp.zeros((1, E), jnp.float32))

    nb = jnp.floor((counts + (TB - 1)) / TB)
    lt = (lax.broadcasted_iota(jnp.int32, (E, E), 0) <
          lax.broadcasted_iota(jnp.int32, (E, E), 1)).astype(jnp.float32)
    bs = jnp.dot(nb, lt, preferred_element_type=jnp.float32)
    na = (bs[:, E - 1:] + nb[:, E - 1:]).astype(jnp.int32)
    na_ref[...] = na

    onehot = oh_ref[...]
    csum = cs_ref[...]
    rank = jnp.sum(onehot * (csum - 1.0), axis=1, keepdims=True)
    poff = jnp.sum(onehot * (bs * TB), axis=1, keepdims=True)
    pos_ref[...] = (rank + poff).astype(jnp.int32)

    g_iota = lax.broadcasted_iota(jnp.int32, (G, 1), 0)
    g_eff = jnp.minimum(g_iota, na - 1)
    bs_i = bs.astype(jnp.int32)
    eb_ref[...] = jnp.sum((bs_i <= g_eff).astype(jnp.int32),
                          axis=1, keepdims=True) - 1


def _run_router(x_flat, gate_W, gate_b):
    return pl.pallas_call(
        _router_body,
        out_shape=[
            jax.ShapeDtypeStruct((T, 1), jnp.int32),
            jax.ShapeDtypeStruct((G, 1), jnp.int32),
            jax.ShapeDtypeStruct((1, 1), jnp.int32),
            jax.ShapeDtypeStruct((NP, T, 128), jnp.float32),
        ],
        in_specs=[
            pl.BlockSpec((T, D), lambda: (0, 0)),
            pl.BlockSpec((D, E), lambda: (0, 0)),
            pl.BlockSpec((1, E), lambda: (0, 0)),
        ],
        out_specs=[
            pl.BlockSpec((T, 1), lambda: (0, 0)),
            pl.BlockSpec((G, 1), lambda: (0, 0)),
            pl.BlockSpec((1, 1), lambda: (0, 0)),
            pl.BlockSpec((NP, T, 128), lambda: (0, 0, 0)),
        ],
        scratch_shapes=[
            pltpu.VMEM((T, E), jnp.float32),
            pltpu.VMEM((T, E), jnp.float32),
        ],
    )(x_flat, gate_W, gate_b)


def _ffn_body(eb_s, na_s, xs_ref, w1_ref, b1_ref, w2_ref, b2_ref, out_ref):
    g = pl.program_id(0)

    @pl.when(g < na_s[0])
    def _():
        h = b1_ref[0].astype(jnp.float32)
        acc = jnp.zeros((TB, FF), jnp.float32)
        for j in range(NP):
            acc = acc + jnp.dot(xs_ref[j], w1_ref[0, j * 128:(j + 1) * 128, :],
                                preferred_element_type=jnp.float32)
        hrelu = jnp.maximum(acc + h, 0.0)
        for j in range(NP):
            out_ref[j] = (jnp.dot(hrelu, w2_ref[0, :, j * 128:(j + 1) * 128],
                                  preferred_element_type=jnp.float32)
                          + b2_ref[0, :, j * 128:(j + 1) * 128])


def _run_ffn(eb, na, xs, W1, b1, W2, b2):
    grid_spec = pltpu.PrefetchScalarGridSpec(
        num_scalar_prefetch=2,
        grid=(G,),
        in_specs=[
            pl.BlockSpec((NP, TB, 128), lambda g, eb_s, na_s: (0, g, 0)),
            pl.BlockSpec((1, D, FF), lambda g, eb_s, na_s: (eb_s[g], 0, 0)),
            pl.BlockSpec((1, 1, FF), lambda g, eb_s, na_s: (eb_s[g], 0, 0)),
            pl.BlockSpec((1, FF, D), lambda g, eb_s, na_s: (eb_s[g], 0, 0)),
            pl.BlockSpec((1, 1, D), lambda g, eb_s, na_s: (eb_s[g], 0, 0)),
        ],
        out_specs=pl.BlockSpec((NP, TB, 128), lambda g, eb_s, na_s: (0, g, 0)),
    )
    return pl.pallas_call(
        _ffn_body,
        grid_spec=grid_spec,
        out_shape=jax.ShapeDtypeStruct((NP, P, 128), jnp.float32),
    )(eb, na, xs, W1, b1.reshape(E, 1, FF), W2, b2.reshape(E, 1, D))


@functools.lru_cache(maxsize=None)
def _sc_kernels():
    mesh = plsc.VectorSubcoreMesh(core_axis_name="c", subcore_axis_name="s")

    @functools.partial(
        pl.kernel, mesh=mesh,
        out_type=jax.ShapeDtypeStruct((NP, P, 128), jnp.float32),
        scratch_types=[
            pltpu.VMEM((CHK,), jnp.int32),
            pltpu.VMEM((CHK, 128), jnp.float32),
            pltpu.SemaphoreType.DMA,
        ],
    )
    def sc_dispatch(xp_hbm, pos_hbm, xs_hbm, idx_v, rows_v, sem):
        wid = lax.axis_index("s") * 2 + lax.axis_index("c")
        base = wid * CHK
        pltpu.sync_copy(pos_hbm.at[pl.ds(base, CHK)], idx_v)
        for j in range(NP):
            pltpu.sync_copy(xp_hbm.at[j, pl.ds(base, CHK)], rows_v)
            pltpu.async_copy(rows_v, xs_hbm.at[j].at[idx_v], sem).wait()

    @functools.partial(
        pl.kernel, mesh=mesh,
        out_type=jax.ShapeDtypeStruct((NP, T, 128), jnp.float32),
        scratch_types=[
            pltpu.VMEM((CHK,), jnp.int32),
            pltpu.VMEM((CHK, 128), jnp.float32),
            pltpu.SemaphoreType.DMA,
        ],
    )
    def sc_combine(ys_hbm, pos_hbm, out_hbm, idx_v, rows_v, sem):
        wid = lax.axis_index("s") * 2 + lax.axis_index("c")
        base = wid * CHK
        pltpu.sync_copy(pos_hbm.at[pl.ds(base, CHK)], idx_v)
        for j in range(NP):
            pltpu.async_copy(ys_hbm.at[j].at[idx_v], rows_v, sem).wait()
            pltpu.sync_copy(rows_v, out_hbm.at[j, pl.ds(base, CHK)])

    return sc_dispatch, sc_combine


def kernel(x, gate_W, gate_b, W1, b1, W2, b2):
    seq_len, batch, dim = x.shape
    x_flat = x.reshape(T, D)
    pos2, eb2, na2, xp = _run_router(x_flat, gate_W, gate_b.reshape(1, E))
    pos = pos2.reshape(T)
    eb = eb2.reshape(G)
    na = na2.reshape(1)
    sc_dispatch, sc_combine = _sc_kernels()
    xs = sc_dispatch(xp, pos)
    ys = _run_ffn(eb, na, xs, W1, b1, W2, b2)
    op = sc_combine(ys, pos)
    out = jnp.transpose(op, (1, 0, 2)).reshape(T, D)
    return out.reshape(seq_len, batch, dim)


# speedup vs baseline: 1.3967x; 1.3967x over previous
import functools

import jax
import jax.numpy as jnp
from jax import lax
from jax.experimental import pallas as pl
from jax.experimental.pallas import tpu as pltpu
from jax.experimental.pallas import tpu_sc as plsc

T = 2048
D = 768
FF = 1024
E = 16
TB = 256
G = T // TB + E
P = G * TB
CH = 256
NW = 32
CHK = T // NW


def _router_body(x_ref, gw_ref, gb_ref, pos_ref, eb_ref, na_ref,
                 oh_ref, cs_ref):
    x = x_ref[...]
    scores = jnp.dot(x, gw_ref[...],
                     preferred_element_type=jnp.float32) + gb_ref[...]
    e_iota = lax.broadcasted_iota(jnp.int32, (T, E), 1)
    m = jnp.max(scores, axis=1, keepdims=True)
    eid = jnp.min(jnp.where(scores >= m, e_iota, E), axis=1, keepdims=True)
    oh_ref[...] = (e_iota == eid).astype(jnp.float32)

    tri = (lax.broadcasted_iota(jnp.int32, (CH, CH), 0) >=
           lax.broadcasted_iota(jnp.int32, (CH, CH), 1)).astype(jnp.float32)

    def body(i, carry):
        blk = oh_ref[pl.ds(i * CH, CH), :]
        cs = jnp.dot(tri, blk, preferred_element_type=jnp.float32) + carry
        cs_ref[pl.ds(i * CH, CH), :] = cs
        return cs[CH - 1:CH, :]

    counts = lax.fori_loop(0, T // CH, body,
                           jnp.zeros((1, E), jnp.float32))

    nb = jnp.floor((counts + (TB - 1)) / TB)
    lt = (lax.broadcasted_iota(jnp.int32, (E, E), 0) <
          lax.broadcasted_iota(jnp.int32, (E, E), 1)).astype(jnp.float32)
    bs = jnp.dot(nb, lt, preferred_element_type=jnp.float32)
    na = (bs[:, E - 1:] + nb[:, E - 1:]).astype(jnp.int32)
    na_ref[...] = na

    onehot = oh_ref[...]
    csum = cs_ref[...]
    rank = jnp.sum(onehot * (csum - 1.0), axis=1, keepdims=True)
    poff = jnp.sum(onehot * (bs * TB), axis=1, keepdims=True)
    pos_ref[...] = (rank + poff).astype(jnp.int32)

    g_iota = lax.broadcasted_iota(jnp.int32, (G, 1), 0)
    g_eff = jnp.minimum(g_iota, na - 1)
    bs_i = bs.astype(jnp.int32)
    eb_ref[...] = jnp.sum((bs_i <= g_eff).astype(jnp.int32),
                          axis=1, keepdims=True) - 1


def _run_router(x_flat, gate_W, gate_b):
    return pl.pallas_call(
        _router_body,
        out_shape=[
            jax.ShapeDtypeStruct((T, 1), jnp.int32),
            jax.ShapeDtypeStruct((G, 1), jnp.int32),
            jax.ShapeDtypeStruct((1, 1), jnp.int32),
        ],
        in_specs=[
            pl.BlockSpec((T, D), lambda: (0, 0)),
            pl.BlockSpec((D, E), lambda: (0, 0)),
            pl.BlockSpec((1, E), lambda: (0, 0)),
        ],
        out_specs=[
            pl.BlockSpec((T, 1), lambda: (0, 0)),
            pl.BlockSpec((G, 1), lambda: (0, 0)),
            pl.BlockSpec((1, 1), lambda: (0, 0)),
        ],
        scratch_shapes=[
            pltpu.VMEM((T, E), jnp.float32),
            pltpu.VMEM((T, E), jnp.float32),
        ],
    )(x_flat, gate_W, gate_b)


def _ffn_body(eb_s, na_s, xs_ref, w1_ref, b1_ref, w2_ref, b2_ref, out_ref):
    g = pl.program_id(0)

    @pl.when(g < na_s[0])
    def _():
        xb = xs_ref[...]
        h = jnp.maximum(
            jnp.dot(xb, w1_ref[0], preferred_element_type=jnp.float32)
            + b1_ref[0], 0.0)
        out_ref[...] = (jnp.dot(h, w2_ref[0],
                                preferred_element_type=jnp.float32)
                        + b2_ref[0])


def _run_ffn(eb, na, xs, W1, b1, W2, b2):
    grid_spec = pltpu.PrefetchScalarGridSpec(
        num_scalar_prefetch=2,
        grid=(G,),
        in_specs=[
            pl.BlockSpec((TB, D), lambda g, eb_s, na_s: (g, 0)),
            pl.BlockSpec((1, D, FF), lambda g, eb_s, na_s: (eb_s[g], 0, 0)),
            pl.BlockSpec((1, 1, FF), lambda g, eb_s, na_s: (eb_s[g], 0, 0)),
            pl.BlockSpec((1, FF, D), lambda g, eb_s, na_s: (eb_s[g], 0, 0)),
            pl.BlockSpec((1, 1, D), lambda g, eb_s, na_s: (eb_s[g], 0, 0)),
        ],
        out_specs=pl.BlockSpec((TB, D), lambda g, eb_s, na_s: (g, 0)),
    )
    return pl.pallas_call(
        _ffn_body,
        grid_spec=grid_spec,
        out_shape=jax.ShapeDtypeStruct((P, D), jnp.float32),
    )(eb, na, xs, W1, b1.reshape(E, 1, FF), W2, b2.reshape(E, 1, D))


@functools.lru_cache(maxsize=None)
def _sc_kernels():
    mesh = plsc.VectorSubcoreMesh(core_axis_name="c", subcore_axis_name="s")

    @functools.partial(
        pl.kernel, mesh=mesh,
        out_type=jax.ShapeDtypeStruct((P, D), jnp.float32),
        scratch_types=[
            pltpu.VMEM((CHK,), jnp.int32),
            pltpu.VMEM((CHK, D), jnp.float32),
            pltpu.SemaphoreType.DMA,
        ],
    )
    def sc_dispatch(x_hbm, pos_hbm, xs_hbm, idx_v, rows_v, sem):
        wid = lax.axis_index("s") * 2 + lax.axis_index("c")
        base = wid * CHK
        pltpu.sync_copy(pos_hbm.at[pl.ds(base, CHK)], idx_v)
        pltpu.sync_copy(x_hbm.at[pl.ds(base, CHK)], rows_v)
        pltpu.async_copy(rows_v, xs_hbm.at[idx_v], sem).wait()

    @functools.partial(
        pl.kernel, mesh=mesh,
        out_type=jax.ShapeDtypeStruct((T, 1, D), jnp.float32),
        scratch_types=[
            pltpu.VMEM((CHK,), jnp.int32),
            pltpu.VMEM((CHK, D), jnp.float32),
            pltpu.SemaphoreType.DMA,
        ],
    )
    def sc_combine(ys_hbm, pos_hbm, out_hbm, idx_v, rows_v, sem):
        wid = lax.axis_index("s") * 2 + lax.axis_index("c")
        base = wid * CHK
        pltpu.sync_copy(pos_hbm.at[pl.ds(base, CHK)], idx_v)
        pltpu.async_copy(ys_hbm.at[idx_v], rows_v, sem).wait()
        pltpu.sync_copy(rows_v, out_hbm.at[pl.ds(base, CHK), 0])

    return sc_dispatch, sc_combine


def kernel(x, gate_W, gate_b, W1, b1, W2, b2):
    seq_len, batch, dim = x.shape
    x_flat = x.reshape(T, D)
    pos2, eb2, na2 = _run_router(x_flat, gate_W, gate_b.reshape(1, E))
    pos = pos2.reshape(T)
    eb = eb2.reshape(G)
    na = na2.reshape(1)
    sc_dispatch, sc_combine = _sc_kernels()
    xs = sc_dispatch(x_flat, pos)
    ys = _run_ffn(eb, na, xs, W1, b1, W2, b2)
    out = sc_combine(ys, pos)
    return out.reshape(seq_len, batch, dim)
